# baseline (speedup 1.0000x reference)
import jax
import jax.numpy as jnp
from jax.experimental import pallas as pl
from jax.experimental.pallas import tpu as pltpu

_N = 256
_TI = 64


def _gat_layer(h, Wl, Wr, a_row, b_row):
    hl = jnp.dot(h, Wl, preferred_element_type=jnp.float32)
    hr = jnp.dot(h, Wr, preferred_element_type=jnp.float32)
    rows = []
    for i0 in range(0, _N, _TI):
        e = hl[i0:i0 + _TI, None, :] + hr[None, :, :]
        e = jnp.where(e >= 0.0, e, 0.2 * e)
        rows.append(jnp.sum(e * a_row[None, :, :], axis=-1))
    scores = jnp.concatenate(rows, axis=0)
    m = jnp.max(scores, axis=-1, keepdims=True)
    p = jnp.exp(scores - m)
    alpha = p / jnp.sum(p, axis=-1, keepdims=True)
    out = jnp.dot(alpha, hr, preferred_element_type=jnp.float32) + b_row
    return out


def _fused_kernel(x_ref, w1l_ref, w1r_ref, a1_ref, b1_ref,
                  w2l_ref, w2r_ref, a2_ref, b2_ref, out_ref):
    x = x_ref[0]
    h = jnp.tanh(_gat_layer(x, w1l_ref[0], w1r_ref[0], a1_ref[0], b1_ref[0]))
    h = jnp.tanh(_gat_layer(h, w2l_ref[0], w2r_ref[0], a2_ref[0], b2_ref[0]))
    out_ref[0, 0] = jnp.mean(h, axis=0, keepdims=True)


def kernel(features, p1_Wl, p1_Wr, p1_a, p1_b, p2_Wl, p2_Wr, p2_a, p2_b,
           v1_Wl, v1_Wr, v1_a, v1_b, v2_Wl, v2_Wr, v2_a, v2_b):
    B, N, F = features.shape
    D = p1_Wl.shape[1]
    w1l = jnp.stack([p1_Wl, v1_Wl])
    w1r = jnp.stack([p1_Wr, v1_Wr])
    a1 = jnp.stack([p1_a, v1_a])[:, None, :]
    b1 = jnp.stack([p1_b, v1_b])[:, None, :]
    w2l = jnp.stack([p2_Wl, v2_Wl])
    w2r = jnp.stack([p2_Wr, v2_Wr])
    a2 = jnp.stack([p2_a, v2_a])[:, None, :]
    b2 = jnp.stack([p2_b, v2_b])[:, None, :]

    out = pl.pallas_call(
        _fused_kernel,
        grid=(2, B),
        in_specs=[
            pl.BlockSpec((1, N, F), lambda n, b: (b, 0, 0)),
            pl.BlockSpec((1, F, D), lambda n, b: (n, 0, 0)),
            pl.BlockSpec((1, F, D), lambda n, b: (n, 0, 0)),
            pl.BlockSpec((1, 1, D), lambda n, b: (n, 0, 0)),
            pl.BlockSpec((1, 1, D), lambda n, b: (n, 0, 0)),
            pl.BlockSpec((1, D, D), lambda n, b: (n, 0, 0)),
            pl.BlockSpec((1, D, D), lambda n, b: (n, 0, 0)),
            pl.BlockSpec((1, 1, D), lambda n, b: (n, 0, 0)),
            pl.BlockSpec((1, 1, D), lambda n, b: (n, 0, 0)),
        ],
        out_specs=pl.BlockSpec((1, 1, 1, D), lambda n, b: (n, b, 0, 0)),
        out_shape=jax.ShapeDtypeStruct((2, B, 1, D), jnp.float32),
        compiler_params=pltpu.CompilerParams(
            dimension_semantics=("parallel", "parallel")),
    )(features, w1l, w1r, a1, b1, w2l, w2r, a2, b2)
    out = out.reshape(2, B, D)
    return out[0], out[1]


# speedup vs baseline: 2.3947x; 2.3947x over previous
import jax
import jax.numpy as jnp
from jax.experimental import pallas as pl
from jax.experimental.pallas import tpu as pltpu

_N = 256
_TI = 64


def _gat_layer(h, Wla, Wra, s04_col, Wr, b_row):
    hlp = jnp.dot(h, Wla, preferred_element_type=jnp.float32)
    hrp = jnp.dot(h, Wra, preferred_element_type=jnp.float32)
    hr = jnp.dot(h, Wr, preferred_element_type=jnp.float32)
    sl = jnp.sum(hlp, axis=1, keepdims=True)
    hrpT = hrp.T
    srT = jnp.sum(hrpT, axis=0, keepdims=True)
    s04 = s04_col[None, :, :]
    rows = []
    for i0 in range(0, _N, _TI):
        u = hlp[i0:i0 + _TI, :, None] + hrpT[None, :, :]
        rows.append(jnp.sum(jnp.abs(u) * s04, axis=1))
    scores = 0.6 * (sl + srT) + jnp.concatenate(rows, axis=0)
    m = jnp.max(scores, axis=-1, keepdims=True)
    p = jnp.exp(scores - m)
    alpha = p / jnp.sum(p, axis=-1, keepdims=True)
    out = jnp.dot(alpha, hr, preferred_element_type=jnp.float32) + b_row
    return out


def _fused_kernel(x_ref, w1la_ref, w1ra_ref, s1_ref, w1r_ref, b1_ref,
                  w2la_ref, w2ra_ref, s2_ref, w2r_ref, b2_ref, out_ref):
    x = x_ref[0]
    h = jnp.tanh(_gat_layer(x, w1la_ref[0], w1ra_ref[0], s1_ref[0],
                            w1r_ref[0], b1_ref[0]))
    h = jnp.tanh(_gat_layer(h, w2la_ref[0], w2ra_ref[0], s2_ref[0],
                            w2r_ref[0], b2_ref[0]))
    out_ref[0, 0] = jnp.mean(h, axis=0, keepdims=True)


def kernel(features, p1_Wl, p1_Wr, p1_a, p1_b, p2_Wl, p2_Wr, p2_a, p2_b,
           v1_Wl, v1_Wr, v1_a, v1_b, v2_Wl, v2_Wr, v2_a, v2_b):
    B, N, F = features.shape
    D = p1_Wl.shape[1]

    def prep(Wl, Wr, a, b):
        return (Wl * a[None, :], Wr * a[None, :],
                (0.4 * jnp.sign(a))[:, None], Wr, b[None, :])

    l1 = [prep(p1_Wl, p1_Wr, p1_a, p1_b), prep(v1_Wl, v1_Wr, v1_a, v1_b)]
    l2 = [prep(p2_Wl, p2_Wr, p2_a, p2_b), prep(v2_Wl, v2_Wr, v2_a, v2_b)]
    w1la, w1ra, s1, w1r, b1 = (jnp.stack([t[k] for t in l1]) for k in range(5))
    w2la, w2ra, s2, w2r, b2 = (jnp.stack([t[k] for t in l2]) for k in range(5))

    out = pl.pallas_call(
        _fused_kernel,
        grid=(2, B),
        in_specs=[
            pl.BlockSpec((1, N, F), lambda n, b: (b, 0, 0)),
            pl.BlockSpec((1, F, D), lambda n, b: (n, 0, 0)),
            pl.BlockSpec((1, F, D), lambda n, b: (n, 0, 0)),
            pl.BlockSpec((1, D, 1), lambda n, b: (n, 0, 0)),
            pl.BlockSpec((1, F, D), lambda n, b: (n, 0, 0)),
            pl.BlockSpec((1, 1, D), lambda n, b: (n, 0, 0)),
            pl.BlockSpec((1, D, D), lambda n, b: (n, 0, 0)),
            pl.BlockSpec((1, D, D), lambda n, b: (n, 0, 0)),
            pl.BlockSpec((1, D, 1), lambda n, b: (n, 0, 0)),
            pl.BlockSpec((1, D, D), lambda n, b: (n, 0, 0)),
            pl.BlockSpec((1, 1, D), lambda n, b: (n, 0, 0)),
        ],
        out_specs=pl.BlockSpec((1, 1, 1, D), lambda n, b: (n, b, 0, 0)),
        out_shape=jax.ShapeDtypeStruct((2, B, 1, D), jnp.float32),
        compiler_params=pltpu.CompilerParams(
            dimension_semantics=("parallel", "parallel")),
    )(features, w1la, w1ra, s1, w1r, b1, w2la, w2ra, s2, w2r, b2)
    out = out.reshape(2, B, D)
    return out[0], out[1]


# speedup vs baseline: 2.7162x; 1.1343x over previous
import jax
import jax.numpy as jnp
from jax.experimental import pallas as pl
from jax.experimental.pallas import tpu as pltpu

_N = 256
_TI = 64


def _gat_layer(h, Wla, Wra, s04_col, Wr, b_row):
    hlp = jnp.dot(h, Wla, preferred_element_type=jnp.float32)
    hrp = jnp.dot(h, Wra, preferred_element_type=jnp.float32)
    hr = jnp.dot(h, Wr, preferred_element_type=jnp.float32)
    sl = jnp.sum(hlp, axis=1, keepdims=True)
    hrpT = hrp.T
    srT = jnp.sum(hrpT, axis=0, keepdims=True)
    hlp16 = hlp.astype(jnp.bfloat16)
    hrpT16 = hrpT.astype(jnp.bfloat16)
    s04 = s04_col[None, :, :].astype(jnp.bfloat16)
    rows = []
    for i0 in range(0, _N, _TI):
        u = hlp16[i0:i0 + _TI, :, None] + hrpT16[None, :, :]
        t = jnp.sum(jnp.abs(u) * s04, axis=1)
        rows.append(t.astype(jnp.float32))
    scores = 0.6 * (sl + srT) + jnp.concatenate(rows, axis=0)
    m = jnp.max(scores, axis=-1, keepdims=True)
    p = jnp.exp(scores - m)
    alpha = p / jnp.sum(p, axis=-1, keepdims=True)
    out = jnp.dot(alpha, hr, preferred_element_type=jnp.float32) + b_row
    return out


def _fused_kernel(x_ref, w1la_ref, w1ra_ref, s1_ref, w1r_ref, b1_ref,
                  w2la_ref, w2ra_ref, s2_ref, w2r_ref, b2_ref, out_ref):
    x = x_ref[0]
    h = jnp.tanh(_gat_layer(x, w1la_ref[0], w1ra_ref[0], s1_ref[0],
                            w1r_ref[0], b1_ref[0]))
    h = jnp.tanh(_gat_layer(h, w2la_ref[0], w2ra_ref[0], s2_ref[0],
                            w2r_ref[0], b2_ref[0]))
    out_ref[0, 0] = jnp.mean(h, axis=0, keepdims=True)


def kernel(features, p1_Wl, p1_Wr, p1_a, p1_b, p2_Wl, p2_Wr, p2_a, p2_b,
           v1_Wl, v1_Wr, v1_a, v1_b, v2_Wl, v2_Wr, v2_a, v2_b):
    B, N, F = features.shape
    D = p1_Wl.shape[1]

    def prep(Wl, Wr, a, b):
        return (Wl * a[None, :], Wr * a[None, :],
                (0.4 * jnp.sign(a))[:, None], Wr, b[None, :])

    l1 = [prep(p1_Wl, p1_Wr, p1_a, p1_b), prep(v1_Wl, v1_Wr, v1_a, v1_b)]
    l2 = [prep(p2_Wl, p2_Wr, p2_a, p2_b), prep(v2_Wl, v2_Wr, v2_a, v2_b)]
    w1la, w1ra, s1, w1r, b1 = (jnp.stack([t[k] for t in l1]) for k in range(5))
    w2la, w2ra, s2, w2r, b2 = (jnp.stack([t[k] for t in l2]) for k in range(5))

    out = pl.pallas_call(
        _fused_kernel,
        grid=(2, B),
        in_specs=[
            pl.BlockSpec((1, N, F), lambda n, b: (b, 0, 0)),
            pl.BlockSpec((1, F, D), lambda n, b: (n, 0, 0)),
            pl.BlockSpec((1, F, D), lambda n, b: (n, 0, 0)),
            pl.BlockSpec((1, D, 1), lambda n, b: (n, 0, 0)),
            pl.BlockSpec((1, F, D), lambda n, b: (n, 0, 0)),
            pl.BlockSpec((1, 1, D), lambda n, b: (n, 0, 0)),
            pl.BlockSpec((1, D, D), lambda n, b: (n, 0, 0)),
            pl.BlockSpec((1, D, D), lambda n, b: (n, 0, 0)),
            pl.BlockSpec((1, D, 1), lambda n, b: (n, 0, 0)),
            pl.BlockSpec((1, D, D), lambda n, b: (n, 0, 0)),
            pl.BlockSpec((1, 1, D), lambda n, b: (n, 0, 0)),
        ],
        out_specs=pl.BlockSpec((1, 1, 1, D), lambda n, b: (n, b, 0, 0)),
        out_shape=jax.ShapeDtypeStruct((2, B, 1, D), jnp.float32),
        compiler_params=pltpu.CompilerParams(
            dimension_semantics=("parallel", "parallel")),
    )(features, w1la, w1ra, s1, w1r, b1, w2la, w2ra, s2, w2r, b2)
    out = out.reshape(2, B, D)
    return out[0], out[1]


# speedup vs baseline: 3.3779x; 1.2436x over previous
import jax
import jax.numpy as jnp
from jax.experimental import pallas as pl
from jax.experimental.pallas import tpu as pltpu

_N = 256
_TI = 64


def _gat_layer(h, Wla, Wra, s04_col, Wr, b_row):
    hlp = jnp.dot(h, Wla, preferred_element_type=jnp.float32)
    hrp = jnp.dot(h, Wra, preferred_element_type=jnp.float32)
    hr = jnp.dot(h, Wr, preferred_element_type=jnp.float32)
    sl = jnp.sum(hlp, axis=1, keepdims=True)
    hrpT = hrp.T
    srT = jnp.sum(hrpT, axis=0, keepdims=True)
    hlp16 = hlp.astype(jnp.bfloat16)
    hrpT16 = hrpT.astype(jnp.bfloat16)
    s04 = s04_col[None, :, :].astype(jnp.bfloat16)
    rows = []
    for i0 in range(0, _N, _TI):
        u = hlp16[i0:i0 + _TI, :, None] + hrpT16[None, :, :]
        t = jnp.abs(u) * s04
        t = t[:, :32, :] + t[:, 32:, :]
        t = t[:, :16, :] + t[:, 16:, :]
        rows.append(jnp.sum(t.astype(jnp.float32), axis=1))
    scores = 0.6 * (sl + srT) + jnp.concatenate(rows, axis=0)
    m = jnp.max(scores, axis=-1, keepdims=True)
    p = jnp.exp(scores - m)
    alpha = p / jnp.sum(p, axis=-1, keepdims=True)
    out = jnp.dot(alpha, hr, preferred_element_type=jnp.float32) + b_row
    return out


def _fused_kernel(x_ref, w1la_ref, w1ra_ref, s1_ref, w1r_ref, b1_ref,
                  w2la_ref, w2ra_ref, s2_ref, w2r_ref, b2_ref, out_ref):
    x = x_ref[0]
    h = jnp.tanh(_gat_layer(x, w1la_ref[0], w1ra_ref[0], s1_ref[0],
                            w1r_ref[0], b1_ref[0]))
    h = jnp.tanh(_gat_layer(h, w2la_ref[0], w2ra_ref[0], s2_ref[0],
                            w2r_ref[0], b2_ref[0]))
    out_ref[0, 0] = jnp.mean(h, axis=0, keepdims=True)


def kernel(features, p1_Wl, p1_Wr, p1_a, p1_b, p2_Wl, p2_Wr, p2_a, p2_b,
           v1_Wl, v1_Wr, v1_a, v1_b, v2_Wl, v2_Wr, v2_a, v2_b):
    B, N, F = features.shape
    D = p1_Wl.shape[1]

    def prep(Wl, Wr, a, b):
        return (Wl * a[None, :], Wr * a[None, :],
                (0.4 * jnp.sign(a))[:, None], Wr, b[None, :])

    l1 = [prep(p1_Wl, p1_Wr, p1_a, p1_b), prep(v1_Wl, v1_Wr, v1_a, v1_b)]
    l2 = [prep(p2_Wl, p2_Wr, p2_a, p2_b), prep(v2_Wl, v2_Wr, v2_a, v2_b)]
    w1la, w1ra, s1, w1r, b1 = (jnp.stack([t[k] for t in l1]) for k in range(5))
    w2la, w2ra, s2, w2r, b2 = (jnp.stack([t[k] for t in l2]) for k in range(5))

    out = pl.pallas_call(
        _fused_kernel,
        grid=(2, B),
        in_specs=[
            pl.BlockSpec((1, N, F), lambda n, b: (b, 0, 0)),
            pl.BlockSpec((1, F, D), lambda n, b: (n, 0, 0)),
            pl.BlockSpec((1, F, D), lambda n, b: (n, 0, 0)),
            pl.BlockSpec((1, D, 1), lambda n, b: (n, 0, 0)),
            pl.BlockSpec((1, F, D), lambda n, b: (n, 0, 0)),
            pl.BlockSpec((1, 1, D), lambda n, b: (n, 0, 0)),
            pl.BlockSpec((1, D, D), lambda n, b: (n, 0, 0)),
            pl.BlockSpec((1, D, D), lambda n, b: (n, 0, 0)),
            pl.BlockSpec((1, D, 1), lambda n, b: (n, 0, 0)),
            pl.BlockSpec((1, D, D), lambda n, b: (n, 0, 0)),
            pl.BlockSpec((1, 1, D), lambda n, b: (n, 0, 0)),
        ],
        out_specs=pl.BlockSpec((1, 1, 1, D), lambda n, b: (n, b, 0, 0)),
        out_shape=jax.ShapeDtypeStruct((2, B, 1, D), jnp.float32),
        compiler_params=pltpu.CompilerParams(
            dimension_semantics=("parallel", "parallel")),
    )(features, w1la, w1ra, s1, w1r, b1, w2la, w2ra, s2, w2r, b2)
    out = out.reshape(2, B, D)
    return out[0], out[1]


# speedup vs baseline: 3.9695x; 1.1751x over previous
import jax
import jax.numpy as jnp
from jax.experimental import pallas as pl
from jax.experimental.pallas import tpu as pltpu

_N = 256
_TI = 64


def _gat_layer(h, Wla, Wra, s04_col, Wr, b_row, bd):
    hlp = jnp.dot(h, Wla, preferred_element_type=jnp.float32)
    hrp = jnp.dot(h, Wra, preferred_element_type=jnp.float32)
    hr = jnp.dot(h, Wr, preferred_element_type=jnp.float32)
    sl = jnp.sum(hlp, axis=1, keepdims=True)
    hrpT = hrp.T
    srT = jnp.sum(hrpT, axis=0, keepdims=True)
    hlp16 = hlp.astype(jnp.bfloat16)
    hrpT16 = hrpT.astype(jnp.bfloat16)
    s04 = s04_col[None, :, :].astype(jnp.bfloat16)
    rows = []
    for i0 in range(0, _N, _TI):
        u = hlp16[i0:i0 + _TI, :, None] + hrpT16[None, :, :]
        t = jnp.abs(u) * s04
        t = t[:, :32, :] + t[:, 32:, :]
        t = t[:, :16, :] + t[:, 16:, :]
        rows.append(jnp.dot(bd, t.reshape(_TI * 16, _N),
                            preferred_element_type=jnp.float32))
    scores = 0.6 * (sl + srT) + jnp.concatenate(rows, axis=0)
    m = jnp.max(scores, axis=-1, keepdims=True)
    p = jnp.exp(scores - m)
    alpha = p / jnp.sum(p, axis=-1, keepdims=True)
    out = jnp.dot(alpha, hr, preferred_element_type=jnp.float32) + b_row
    return out


def _fused_kernel(x_ref, w1la_ref, w1ra_ref, s1_ref, w1r_ref, b1_ref,
                  w2la_ref, w2ra_ref, s2_ref, w2r_ref, b2_ref, bd_ref,
                  out_ref):
    x = x_ref[0]
    bd = bd_ref[:]
    h = jnp.tanh(_gat_layer(x, w1la_ref[0], w1ra_ref[0], s1_ref[0],
                            w1r_ref[0], b1_ref[0], bd))
    h = jnp.tanh(_gat_layer(h, w2la_ref[0], w2ra_ref[0], s2_ref[0],
                            w2r_ref[0], b2_ref[0], bd))
    out_ref[0, 0] = jnp.mean(h, axis=0, keepdims=True)


def kernel(features, p1_Wl, p1_Wr, p1_a, p1_b, p2_Wl, p2_Wr, p2_a, p2_b,
           v1_Wl, v1_Wr, v1_a, v1_b, v2_Wl, v2_Wr, v2_a, v2_b):
    B, N, F = features.shape
    D = p1_Wl.shape[1]

    def prep(Wl, Wr, a, b):
        return (Wl * a[None, :], Wr * a[None, :],
                (0.4 * jnp.sign(a))[:, None], Wr, b[None, :])

    bd = (jnp.arange(_TI)[:, None] ==
          (jnp.arange(_TI * 16) // 16)[None, :]).astype(jnp.bfloat16)

    l1 = [prep(p1_Wl, p1_Wr, p1_a, p1_b), prep(v1_Wl, v1_Wr, v1_a, v1_b)]
    l2 = [prep(p2_Wl, p2_Wr, p2_a, p2_b), prep(v2_Wl, v2_Wr, v2_a, v2_b)]
    w1la, w1ra, s1, w1r, b1 = (jnp.stack([t[k] for t in l1]) for k in range(5))
    w2la, w2ra, s2, w2r, b2 = (jnp.stack([t[k] for t in l2]) for k in range(5))

    out = pl.pallas_call(
        _fused_kernel,
        grid=(2, B),
        in_specs=[
            pl.BlockSpec((1, N, F), lambda n, b: (b, 0, 0)),
            pl.BlockSpec((1, F, D), lambda n, b: (n, 0, 0)),
            pl.BlockSpec((1, F, D), lambda n, b: (n, 0, 0)),
            pl.BlockSpec((1, D, 1), lambda n, b: (n, 0, 0)),
            pl.BlockSpec((1, F, D), lambda n, b: (n, 0, 0)),
            pl.BlockSpec((1, 1, D), lambda n, b: (n, 0, 0)),
            pl.BlockSpec((1, D, D), lambda n, b: (n, 0, 0)),
            pl.BlockSpec((1, D, D), lambda n, b: (n, 0, 0)),
            pl.BlockSpec((1, D, 1), lambda n, b: (n, 0, 0)),
            pl.BlockSpec((1, D, D), lambda n, b: (n, 0, 0)),
            pl.BlockSpec((1, 1, D), lambda n, b: (n, 0, 0)),
            pl.BlockSpec((_TI, _TI * 16), lambda n, b: (0, 0)),
        ],
        out_specs=pl.BlockSpec((1, 1, 1, D), lambda n, b: (n, b, 0, 0)),
        out_shape=jax.ShapeDtypeStruct((2, B, 1, D), jnp.float32),
        compiler_params=pltpu.CompilerParams(
            dimension_semantics=("parallel", "parallel")),
    )(features, w1la, w1ra, s1, w1r, b1, w2la, w2ra, s2, w2r, b2, bd)
    out = out.reshape(2, B, D)
    return out[0], out[1]
